# baseline (speedup 1.0000x reference)
import jax
import jax.numpy as jnp
from jax import lax
from jax.experimental import pallas as pl
from jax.experimental.pallas import tpu as pltpu
from jax.experimental.pallas import tpu_sc as plsc

N = 10000
NPAD = 10240
E = 320000
D = 128
P = 4096

NC = 2
NS = 16
NW = NC * NS
L = 16

EPT = E // NW
CH = 128
CHR = 125
NCHUNK = EPT // CHR
RPT = NPAD // NS
PPT = P // NW
BR = 2048

_F32 = jnp.float32


def _mk_mesh():
    return plsc.VectorSubcoreMesh(
        core_axis_name="c", subcore_axis_name="s", num_cores=NC, num_subcores=NS
    )


def _agg_body(with_counts):
    def body(feat, src3, dst3, zeros, *rest):
        if with_counts:
            (psum, cnts, acc, sidx_a, didx_a, sidx_b, didx_b,
             rows_a, rows_b, cnt_v, sa, sb, ia, ib) = rest
        else:
            (psum, acc, sidx_a, didx_a, sidx_b, didx_b,
             rows_a, rows_b, sa, sb, ia, ib) = rest
        c = lax.axis_index("c")
        s = lax.axis_index("s")
        wid = s * NC + c

        if with_counts:
            z16 = jnp.zeros((L,), _F32)

            def zloop(i, carry):
                cnt_v[pl.ds(i * L, L)] = z16
                return carry

            lax.fori_loop(0, NPAD // L, zloop, 0)

        pltpu.sync_copy(zeros.at[pl.ds(s * RPT, RPT)], acc.at[pl.ds(s * RPT, RPT)])
        plsc.subcore_barrier()

        ones16 = jnp.ones((L,), _F32)

        def idx_start(i, sbuf, dbuf, sem):
            pltpu.async_copy(src3.at[wid, i], sbuf, sem)
            pltpu.async_copy(dst3.at[wid, i], dbuf, sem)

        def idx_wait(sbuf, dbuf, sem):
            pltpu.make_async_copy(src3.at[wid, 0], sbuf, sem).wait()
            pltpu.make_async_copy(dst3.at[wid, 0], dbuf, sem).wait()

        def gather(sbuf, buf, sem):
            return pltpu.async_copy(feat.at[sbuf], buf, sem)

        def gwait(buf, sem):
            pltpu.make_async_copy(feat.at[sidx_a], buf, sem).wait()

        def scat(buf, dbuf):
            pltpu.sync_copy(buf, acc.at[dbuf], add=True)

        def hist(dbuf):
            if with_counts:
                for j in range(CH // L):
                    idx = dbuf[pl.ds(j * L, L)]
                    plsc.addupdate_scatter(cnt_v, [idx], ones16)

        idx_start(0, sidx_a, didx_a, ia)
        idx_wait(sidx_a, didx_a, ia)
        gather(sidx_a, rows_a, sa)
        idx_start(1, sidx_b, didx_b, ib)

        def step(k, carry):
            i0 = 2 * k
            gwait(rows_a, sa)
            idx_wait(sidx_b, didx_b, ib)
            gather(sidx_b, rows_b, sb)
            scat(rows_a, didx_a)
            hist(didx_a)
            idx_start(i0 + 2, sidx_a, didx_a, ia)
            gwait(rows_b, sb)
            idx_wait(sidx_a, didx_a, ia)
            gather(sidx_a, rows_a, sa)
            scat(rows_b, didx_b)
            hist(didx_b)
            idx_start(i0 + 3, sidx_b, didx_b, ib)
            return carry

        lax.fori_loop(0, NCHUNK // 2 - 1, step, 0)
        gwait(rows_a, sa)
        idx_wait(sidx_b, didx_b, ib)
        gather(sidx_b, rows_b, sb)
        scat(rows_a, didx_a)
        hist(didx_a)
        gwait(rows_b, sb)
        scat(rows_b, didx_b)
        hist(didx_b)

        if with_counts:
            pltpu.sync_copy(cnt_v, cnts.at[wid])
        plsc.subcore_barrier()

        pltpu.sync_copy(acc.at[pl.ds(s * RPT, RPT)], psum.at[c, pl.ds(s * RPT, RPT)])

    return body


def _make_agg(with_counts):
    out_type = [jax.ShapeDtypeStruct((NC, NPAD, D), _F32)]
    if with_counts:
        out_type.append(jax.ShapeDtypeStruct((NW, NPAD), _F32))
    scratch = [
        pltpu.VMEM_SHARED((NPAD, D), _F32),
        pltpu.VMEM((CH,), jnp.int32),
        pltpu.VMEM((CH,), jnp.int32),
        pltpu.VMEM((CH,), jnp.int32),
        pltpu.VMEM((CH,), jnp.int32),
        pltpu.VMEM((CH, D), _F32),
        pltpu.VMEM((CH, D), _F32),
    ]
    if with_counts:
        scratch.append(pltpu.VMEM((NPAD,), _F32))
    scratch += [pltpu.SemaphoreType.DMA, pltpu.SemaphoreType.DMA,
                pltpu.SemaphoreType.DMA, pltpu.SemaphoreType.DMA]
    return pl.kernel(
        _agg_body(with_counts),
        out_type=out_type,
        mesh=_mk_mesh(),
        compiler_params=pltpu.CompilerParams(needs_layout_passes=False),
        scratch_types=scratch,
    )


def _tc_layer1(ps, cb, ft, wl, wr, bl, oh, orc):
    ssum = ps[0] + ps[1]
    cnt = jnp.sum(cb[...], axis=0).reshape(BR, 1)
    recip = 1.0 / jnp.maximum(cnt, 1.0)
    mean = ssum * recip
    acc = lax.dot_general(mean, wl[...], (((1,), (1,)), ((), ())),
                          preferred_element_type=_F32)
    acc += lax.dot_general(ft[...], wr[...], (((1,), (1,)), ((), ())),
                           preferred_element_type=_F32)
    acc += bl[...]
    oh[...] = jnp.maximum(acc, 0.0)
    orc[...] = jnp.broadcast_to(recip, (BR, 8))


def _tc_layer2(ps, rc, ft, wl, wr, bl, wuv, buv, o):
    mean = (ps[0] + ps[1]) * rc[:, :1]
    h2 = lax.dot_general(mean, wl[...], (((1,), (1,)), ((), ())),
                         preferred_element_type=_F32)
    h2 += lax.dot_general(ft[...], wr[...], (((1,), (1,)), ((), ())),
                          preferred_element_type=_F32)
    h2 += bl[...]
    o[...] = lax.dot_general(h2, wuv[...], (((1,), (0,)), ((), ())),
                             preferred_element_type=_F32) + buv[...]


def _pairs_body(uh, vh, p0, p1, out, u_v, v_v, p0_v, p1_v, out_v):
    c = lax.axis_index("c")
    s = lax.axis_index("s")
    wid = s * NC + c
    pltpu.sync_copy(uh, u_v)
    pltpu.sync_copy(vh, v_v)
    pltpu.sync_copy(p0.at[pl.ds(wid * PPT, PPT)], p0_v)
    pltpu.sync_copy(p1.at[pl.ds(wid * PPT, PPT)], p1_v)
    for j in range(PPT // L):
        i0 = p0_v[pl.ds(j * L, L)]
        i1 = p1_v[pl.ds(j * L, L)]
        u = plsc.load_gather(u_v, [i0])
        v = plsc.load_gather(v_v, [i1])
        z = u + v
        out_v[pl.ds(j * L, L)] = 1.0 / (1.0 + jnp.exp(-z))
    pltpu.sync_copy(out_v, out.at[pl.ds(wid * PPT, PPT)])


def kernel(x, edge_index, pairs, W1l, b1l, W1r, W2l, b2l, W2r, Wlp, blp):
    src = edge_index[0].astype(jnp.int32)
    dst = edge_index[1].astype(jnp.int32)
    p0 = pairs[:, 0].astype(jnp.int32)
    p1 = pairs[:, 1].astype(jnp.int32)

    src3 = jnp.pad(src.reshape(NW, NCHUNK, CHR), ((0, 0), (0, 0), (0, CH - CHR)))
    dst3 = jnp.pad(dst.reshape(NW, NCHUNK, CHR), ((0, 0), (0, 0), (0, CH - CHR)),
                   constant_values=NPAD - 1)

    xp = jnp.pad(x, ((0, NPAD - N), (0, 0)))
    zeros_d = jnp.zeros((NPAD, D), _F32)

    psum1, cnts = _make_agg(True)(xp, src3, dst3, zeros_d)

    wspec = pl.BlockSpec((D, D), lambda i: (0, 0))
    bspec = pl.BlockSpec((1, D), lambda i: (0, 0))
    h, rec = pl.pallas_call(
        _tc_layer1,
        grid=(NPAD // BR,),
        in_specs=[
            pl.BlockSpec((NC, BR, D), lambda i: (0, i, 0)),
            pl.BlockSpec((NW, BR), lambda i: (0, i)),
            pl.BlockSpec((BR, D), lambda i: (i, 0)),
            wspec, wspec, bspec,
        ],
        out_specs=[
            pl.BlockSpec((BR, D), lambda i: (i, 0)),
            pl.BlockSpec((BR, 8), lambda i: (i, 0)),
        ],
        out_shape=[
            jax.ShapeDtypeStruct((NPAD, D), _F32),
            jax.ShapeDtypeStruct((NPAD, 8), _F32),
        ],
    )(psum1, cnts, xp, W1l, W1r, b1l.reshape(1, D))

    (psum2,) = _make_agg(False)(h, src3, dst3, zeros_d)

    wuv = Wlp.reshape(2, D).T
    buv = jnp.concatenate([blp, jnp.zeros((1,), _F32)]).reshape(1, 2)
    uv = pl.pallas_call(
        _tc_layer2,
        grid=(NPAD // BR,),
        in_specs=[
            pl.BlockSpec((NC, BR, D), lambda i: (0, i, 0)),
            pl.BlockSpec((BR, 8), lambda i: (i, 0)),
            pl.BlockSpec((BR, D), lambda i: (i, 0)),
            wspec, wspec, bspec,
            pl.BlockSpec((D, 2), lambda i: (0, 0)),
            pl.BlockSpec((1, 2), lambda i: (0, 0)),
        ],
        out_specs=pl.BlockSpec((BR, 2), lambda i: (i, 0)),
        out_shape=jax.ShapeDtypeStruct((NPAD, 2), _F32),
    )(psum2, rec, h, W2l, W2r, b2l.reshape(1, D), wuv, buv)

    pairk = pl.kernel(
        _pairs_body,
        out_type=jax.ShapeDtypeStruct((P,), _F32),
        mesh=_mk_mesh(),
        compiler_params=pltpu.CompilerParams(needs_layout_passes=False),
        scratch_types=[
            pltpu.VMEM((NPAD,), _F32),
            pltpu.VMEM((NPAD,), _F32),
            pltpu.VMEM((PPT,), jnp.int32),
            pltpu.VMEM((PPT,), jnp.int32),
            pltpu.VMEM((PPT,), _F32),
        ],
    )
    return pairk(uv[:, 0], uv[:, 1], p0, p1)


# speedup vs baseline: 1.3620x; 1.3620x over previous
import jax
import jax.numpy as jnp
from jax import lax
from jax.experimental import pallas as pl
from jax.experimental.pallas import tpu as pltpu
from jax.experimental.pallas import tpu_sc as plsc

N = 10000
NPAD = 10240
E = 320000
D = 128
P = 4096

NC = 2
NS = 16
NW = NC * NS
L = 16

EPT = E // NW
CH = 80
Q = 3
NCHUNK = EPT // CH + 1
WAVES = NCHUNK // Q
RPT = NPAD // NS
PPT = P // NW
BR = 2048

_F32 = jnp.float32


def _mk_mesh():
    return plsc.VectorSubcoreMesh(
        core_axis_name="c", subcore_axis_name="s", num_cores=NC, num_subcores=NS
    )


def _agg_body(with_counts):
    def body(feat, src3, dst3, *rest):
        if with_counts:
            psum, cnts, acc = rest[:3]
            rest = rest[3:]
        else:
            psum, acc = rest[:2]
            rest = rest[2:]
        sidx = rest[0:Q]
        didx = rest[Q:2 * Q]
        rows = rest[2 * Q:3 * Q]
        rest = rest[3 * Q:]
        if with_counts:
            cnt_v = rest[0]
            rest = rest[1:]
        sg = rest[0:Q]
        ss = rest[Q:2 * Q]
        c = lax.axis_index("c")
        s = lax.axis_index("s")
        wid = s * NC + c

        z16 = jnp.zeros((L,), _F32)
        if with_counts:

            def zloop(i, carry):
                cnt_v[pl.ds(i * L, L)] = z16
                return carry

            lax.fori_loop(0, NPAD // L, zloop, 0)

        def zrow(i, carry):
            for j in range(D // L):
                rows[0][i, pl.ds(j * L, L)] = z16
            return carry

        lax.fori_loop(0, CH, zrow, 0)
        for q in range(RPT // CH):
            pltpu.sync_copy(rows[0], acc.at[pl.ds(s * RPT + q * CH, CH)])
        plsc.subcore_barrier()

        ones16 = jnp.ones((L,), _F32)

        def hist(dbuf):
            if with_counts:
                for j in range(CH // L):
                    idx = dbuf[pl.ds(j * L, L)]
                    plsc.addupdate_scatter(cnt_v, [idx], ones16)

        def wave(w, carry):
            i0 = w * Q
            for b in range(Q):
                @pl.when(w > 0)
                def _drain():
                    pltpu.make_async_copy(rows[b], acc.at[didx[b]],
                                          ss[b]).wait()

                pltpu.sync_copy(src3.at[wid, i0 + b], sidx[b])
                pltpu.sync_copy(dst3.at[wid, i0 + b], didx[b])
                pltpu.async_copy(feat.at[sidx[b]], rows[b], sg[b])
                hist(didx[b])
            for b in range(Q):
                pltpu.make_async_copy(feat.at[sidx[b]], rows[b], sg[b]).wait()
                pltpu.async_copy(rows[b], acc.at[didx[b]], ss[b], add=True)
            return carry

        lax.fori_loop(0, WAVES, wave, 0)
        for b in range(Q):
            pltpu.make_async_copy(rows[b], acc.at[didx[b]], ss[b]).wait()

        if with_counts:
            pltpu.sync_copy(cnt_v, cnts.at[wid])
        plsc.subcore_barrier()

        pltpu.sync_copy(acc.at[pl.ds(s * RPT, RPT)], psum.at[c, pl.ds(s * RPT, RPT)])

    return body


def _make_agg(with_counts):
    out_type = [jax.ShapeDtypeStruct((NC, NPAD, D), _F32)]
    if with_counts:
        out_type.append(jax.ShapeDtypeStruct((NW, NPAD), _F32))
    scratch = [pltpu.VMEM_SHARED((NPAD, D), _F32)]
    scratch += [pltpu.VMEM((CH,), jnp.int32) for _ in range(2 * Q)]
    scratch += [pltpu.VMEM((CH, D), _F32) for _ in range(Q)]
    if with_counts:
        scratch.append(pltpu.VMEM((NPAD,), _F32))
    scratch += [pltpu.SemaphoreType.DMA for _ in range(2 * Q)]
    return pl.kernel(
        _agg_body(with_counts),
        out_type=out_type,
        mesh=_mk_mesh(),
        compiler_params=pltpu.CompilerParams(needs_layout_passes=False),
        scratch_types=scratch,
    )


def _tc_layer1(ps, cb, ft, wl, wr, bl, oh, orc):
    ssum = ps[0] + ps[1]
    cnt = jnp.sum(cb[...], axis=0).reshape(BR, 1)
    recip = 1.0 / jnp.maximum(cnt, 1.0)
    mean = ssum * recip
    acc = lax.dot_general(mean, wl[...], (((1,), (1,)), ((), ())),
                          preferred_element_type=_F32)
    acc += lax.dot_general(ft[...], wr[...], (((1,), (1,)), ((), ())),
                           preferred_element_type=_F32)
    acc += bl[...]
    oh[...] = jnp.maximum(acc, 0.0)
    orc[...] = jnp.broadcast_to(recip, (BR, 8))


def _tc_layer2(ps, rc, ft, wl, wr, bl, wuv, buv, o):
    mean = (ps[0] + ps[1]) * rc[:, :1]
    h2 = lax.dot_general(mean, wl[...], (((1,), (1,)), ((), ())),
                         preferred_element_type=_F32)
    h2 += lax.dot_general(ft[...], wr[...], (((1,), (1,)), ((), ())),
                          preferred_element_type=_F32)
    h2 += bl[...]
    o[...] = lax.dot_general(h2, wuv[...], (((1,), (0,)), ((), ())),
                             preferred_element_type=_F32) + buv[...]


def _pairs_body(uh, vh, p0, p1, out, u_v, v_v, p0_v, p1_v, out_v):
    c = lax.axis_index("c")
    s = lax.axis_index("s")
    wid = s * NC + c
    pltpu.sync_copy(uh, u_v)
    pltpu.sync_copy(vh, v_v)
    pltpu.sync_copy(p0.at[pl.ds(wid * PPT, PPT)], p0_v)
    pltpu.sync_copy(p1.at[pl.ds(wid * PPT, PPT)], p1_v)
    for j in range(PPT // L):
        i0 = p0_v[pl.ds(j * L, L)]
        i1 = p1_v[pl.ds(j * L, L)]
        u = plsc.load_gather(u_v, [i0])
        v = plsc.load_gather(v_v, [i1])
        z = u + v
        out_v[pl.ds(j * L, L)] = 1.0 / (1.0 + jnp.exp(-z))
    pltpu.sync_copy(out_v, out.at[pl.ds(wid * PPT, PPT)])


def kernel(x, edge_index, pairs, W1l, b1l, W1r, W2l, b2l, W2r, Wlp, blp):
    src = edge_index[0].astype(jnp.int32)
    dst = edge_index[1].astype(jnp.int32)
    p0 = pairs[:, 0].astype(jnp.int32)
    p1 = pairs[:, 1].astype(jnp.int32)

    src3 = jnp.pad(src.reshape(NW, EPT // CH, CH), ((0, 0), (0, 1), (0, 0)))
    dst3 = jnp.pad(dst.reshape(NW, EPT // CH, CH), ((0, 0), (0, 1), (0, 0)),
                   constant_values=NPAD - 1)

    xp = jnp.pad(x, ((0, NPAD - N), (0, 0)))

    psum1, cnts = _make_agg(True)(xp, src3, dst3)

    wspec = pl.BlockSpec((D, D), lambda i: (0, 0))
    bspec = pl.BlockSpec((1, D), lambda i: (0, 0))
    h, rec = pl.pallas_call(
        _tc_layer1,
        grid=(NPAD // BR,),
        in_specs=[
            pl.BlockSpec((NC, BR, D), lambda i: (0, i, 0)),
            pl.BlockSpec((NW, BR), lambda i: (0, i)),
            pl.BlockSpec((BR, D), lambda i: (i, 0)),
            wspec, wspec, bspec,
        ],
        out_specs=[
            pl.BlockSpec((BR, D), lambda i: (i, 0)),
            pl.BlockSpec((BR, 8), lambda i: (i, 0)),
        ],
        out_shape=[
            jax.ShapeDtypeStruct((NPAD, D), _F32),
            jax.ShapeDtypeStruct((NPAD, 8), _F32),
        ],
    )(psum1, cnts, xp, W1l, W1r, b1l.reshape(1, D))

    (psum2,) = _make_agg(False)(h, src3, dst3)

    wuv = Wlp.reshape(2, D).T
    buv = jnp.concatenate([blp, jnp.zeros((1,), _F32)]).reshape(1, 2)
    uv = pl.pallas_call(
        _tc_layer2,
        grid=(NPAD // BR,),
        in_specs=[
            pl.BlockSpec((NC, BR, D), lambda i: (0, i, 0)),
            pl.BlockSpec((BR, 8), lambda i: (i, 0)),
            pl.BlockSpec((BR, D), lambda i: (i, 0)),
            wspec, wspec, bspec,
            pl.BlockSpec((D, 2), lambda i: (0, 0)),
            pl.BlockSpec((1, 2), lambda i: (0, 0)),
        ],
        out_specs=pl.BlockSpec((BR, 2), lambda i: (i, 0)),
        out_shape=jax.ShapeDtypeStruct((NPAD, 2), _F32),
    )(psum2, rec, h, W2l, W2r, b2l.reshape(1, D), wuv, buv)

    pairk = pl.kernel(
        _pairs_body,
        out_type=jax.ShapeDtypeStruct((P,), _F32),
        mesh=_mk_mesh(),
        compiler_params=pltpu.CompilerParams(needs_layout_passes=False),
        scratch_types=[
            pltpu.VMEM((NPAD,), _F32),
            pltpu.VMEM((NPAD,), _F32),
            pltpu.VMEM((PPT,), jnp.int32),
            pltpu.VMEM((PPT,), jnp.int32),
            pltpu.VMEM((PPT,), _F32),
        ],
    )
    return pairk(uv[:, 0], uv[:, 1], p0, p1)
